# baseline (speedup 1.0000x reference)
import functools

import jax
import jax.numpy as jnp
from jax import lax
from jax.experimental import pallas as pl
from jax.experimental.pallas import tpu as pltpu
from jax.experimental.pallas import tpu_sc as plsc

N = 10000
NP = 10240
E = 320000
NC, NS = 2, 16
B = 50
NR = E // B
RPT = NR // (NC * NS)
NPT = NP // NS
CW = 128
ZR = 128
SUB = 40
NSTG = RPT // SUB
BI = 1000
_F32 = jnp.float32

_mesh = plsc.VectorSubcoreMesh(core_axis_name="c", subcore_axis_name="s")


@functools.partial(
    pl.kernel,
    out_type=jax.ShapeDtypeStruct((NC, NP, 16), _F32),
    mesh=_mesh,
    scratch_types=[
        pltpu.VMEM((RPT, B), jnp.int32),
        pltpu.VMEM((B, 16), _F32),
        pltpu.VMEM((ZR, 16), _F32),
        pltpu.VMEM_SHARED((NP, 16), _F32),
    ],
)
def _deg(dst_hbm, out_hbm, dstv, onesv, zv, acc):
    cid = lax.axis_index("c")
    sid = lax.axis_index("s")
    base = (cid * NS + sid) * RPT
    pltpu.sync_copy(dst_hbm.at[pl.ds(base, RPT)], dstv)

    @pl.loop(0, B)
    def _(r):
        onesv[r, :] = jnp.ones((16,), _F32)

    @pl.loop(0, ZR)
    def _(r):
        zv[r, :] = jnp.zeros((16,), _F32)

    @pl.loop(0, NPT // ZR)
    def _(z):
        pltpu.sync_copy(zv, acc.at[pl.ds(sid * NPT + z * ZR, ZR)])

    plsc.subcore_barrier()

    @pl.loop(0, RPT)
    def _(r):
        pltpu.sync_copy(onesv, acc.at[dstv.at[r]], add=True)

    plsc.subcore_barrier()
    pltpu.sync_copy(acc.at[pl.ds(sid * NPT, NPT)],
                    out_hbm.at[cid, pl.ds(sid * NPT, NPT)])


def _make_agg(n_passes):
    @functools.partial(
        pl.kernel,
        out_type=jax.ShapeDtypeStruct((NC, NP, n_passes * CW), _F32),
        mesh=_mesh,
        scratch_types=[
            pltpu.VMEM((SUB, B), jnp.int32),
            pltpu.VMEM((SUB, B), jnp.int32),
            pltpu.VMEM((B, CW), _F32),
            pltpu.VMEM((B, CW), _F32),
            pltpu.VMEM((B, CW), _F32),
            pltpu.VMEM((B, CW), _F32),
            pltpu.VMEM_SHARED((NP, CW), _F32),
            [pltpu.SemaphoreType.DMA] * 4,
            [pltpu.SemaphoreType.DMA] * 4,
        ],
    )
    def agg(src_hbm, dst_hbm, *rest):
        gs = rest[:n_passes]
        out_hbm = rest[n_passes]
        srcv, dstv, b0, b1, b2, b3, acc, gsem, ssem = rest[n_passes + 1:]
        bufs = (b0, b1, b2, b3)
        cid = lax.axis_index("c")
        sid = lax.axis_index("s")
        base = (cid * NS + sid) * RPT

        for p, g_hbm in enumerate(gs):
            @pl.loop(0, B)
            def _(r):
                @pl.loop(0, CW // 16)
                def _(c):
                    b0[r, pl.ds(c * 16, 16)] = jnp.zeros((16,), _F32)

            for z in range(NPT // B):
                pltpu.sync_copy(b0, acc.at[pl.ds(sid * NPT + z * B, B)])
            pltpu.sync_copy(
                b0.at[pl.ds(0, NPT - (NPT // B) * B)],
                acc.at[pl.ds(sid * NPT + (NPT // B) * B,
                             NPT - (NPT // B) * B)])
            plsc.subcore_barrier()

            def gat(k, b, g_hbm=g_hbm):
                pltpu.async_copy(g_hbm.at[srcv.at[k]], bufs[b], gsem[b])

            def gat_wait(k, b, g_hbm=g_hbm):
                pltpu.make_async_copy(
                    g_hbm.at[srcv.at[k]], bufs[b], gsem[b]).wait()

            def sca(k, b):
                pltpu.async_copy(
                    bufs[b], acc.at[dstv.at[k]], ssem[b], add=True)

            def sca_wait(b):
                pltpu.make_async_copy(
                    bufs[b], acc.at[dstv.at[0]], ssem[b]).wait()

            @pl.loop(0, NSTG)
            def _(s):
                sbase = base + s * SUB
                pltpu.sync_copy(src_hbm.at[pl.ds(sbase, SUB)], srcv)
                pltpu.sync_copy(dst_hbm.at[pl.ds(sbase, SUB)], dstv)
                gat(0, 0)
                gat(1, 1)
                gat_wait(0, 0)
                sca(0, 0)
                gat(2, 2)
                gat_wait(1, 1)
                sca(1, 1)
                gat(3, 3)

                @pl.loop(2, SUB - 2, step=4)
                def _(k):
                    for j in range(4):
                        kk = k + j
                        b = (2 + j) % 4
                        gat_wait(kk, b)
                        sca(kk, b)
                        sca_wait((b + 2) % 4)
                        gat(kk + 2, (b + 2) % 4)

                gat_wait(SUB - 2, (SUB - 2) % 4)
                sca(SUB - 2, (SUB - 2) % 4)
                gat_wait(SUB - 1, (SUB - 1) % 4)
                sca(SUB - 1, (SUB - 1) % 4)
                for b in range(4):
                    sca_wait(b)

            plsc.subcore_barrier()
            pltpu.sync_copy(
                acc.at[pl.ds(sid * NPT, NPT)],
                out_hbm.at[cid, pl.ds(sid * NPT, NPT), pl.ds(p * CW, CW)])

    return agg


_agg1 = _make_agg(1)
_agg2 = _make_agg(2)


def _p_body(degp, x, dinv_o, gpre_o):
    deg = degp[0, :, 0] + degp[1, :, 0] + 1.0
    dinv = lax.rsqrt(deg)
    dinv_o[...] = dinv[:, None]
    gpre_o[...] = x[...] * dinv[:, None]


_p_call = pl.pallas_call(
    _p_body,
    grid=(N // BI,),
    in_specs=[
        pl.BlockSpec((NC, BI, 16), lambda i: (0, i, 0)),
        pl.BlockSpec((BI, 128), lambda i: (i, 0)),
    ],
    out_specs=[
        pl.BlockSpec((BI, 1), lambda i: (i, 0)),
        pl.BlockSpec((BI, 128), lambda i: (i, 0)),
    ],
    out_shape=[
        jax.ShapeDtypeStruct((N, 1), _F32),
        jax.ShapeDtypeStruct((N, 128), _F32),
    ],
)


def _l1_body(aggp, gpre, dinv, w, b, lo_o, hi_o):
    u = (aggp[0] + aggp[1] + gpre[...]) * dinv[...]
    h = jnp.maximum(
        jnp.dot(u, w[...], preferred_element_type=_F32) + b[...], 0.0)
    g2 = h * dinv[...]
    lo_o[...] = g2[:, :CW]
    hi_o[...] = g2[:, CW:]


_l1_call = pl.pallas_call(
    _l1_body,
    grid=(N // BI,),
    in_specs=[
        pl.BlockSpec((NC, BI, 128), lambda i: (0, i, 0)),
        pl.BlockSpec((BI, 128), lambda i: (i, 0)),
        pl.BlockSpec((BI, 1), lambda i: (i, 0)),
        pl.BlockSpec((128, 256), lambda i: (0, 0)),
        pl.BlockSpec((1, 256), lambda i: (0, 0)),
    ],
    out_specs=[
        pl.BlockSpec((BI, CW), lambda i: (i, 0)),
        pl.BlockSpec((BI, CW), lambda i: (i, 0)),
    ],
    out_shape=[
        jax.ShapeDtypeStruct((N, CW), _F32),
        jax.ShapeDtypeStruct((N, CW), _F32),
    ],
)


def _l23_body(aggp, glo, ghi, dinv, w2, b2, w3, lo_o, hi_o):
    gpre = jnp.concatenate([glo[...], ghi[...]], axis=1)
    u = (aggp[0] + aggp[1] + gpre) * dinv[...]
    h2 = jnp.maximum(
        jnp.dot(u, w2[...], preferred_element_type=_F32) + b2[...], 0.0)
    g3 = jnp.dot(h2, w3[...], preferred_element_type=_F32) * dinv[...]
    lo_o[...] = g3[:, :CW]
    hi_o[...] = g3[:, CW:]


_l23_call = pl.pallas_call(
    _l23_body,
    grid=(N // BI,),
    in_specs=[
        pl.BlockSpec((NC, BI, 256), lambda i: (0, i, 0)),
        pl.BlockSpec((BI, CW), lambda i: (i, 0)),
        pl.BlockSpec((BI, CW), lambda i: (i, 0)),
        pl.BlockSpec((BI, 1), lambda i: (i, 0)),
        pl.BlockSpec((256, 512), lambda i: (0, 0)),
        pl.BlockSpec((1, 512), lambda i: (0, 0)),
        pl.BlockSpec((512, 256), lambda i: (0, 0)),
    ],
    out_specs=[
        pl.BlockSpec((BI, CW), lambda i: (i, 0)),
        pl.BlockSpec((BI, CW), lambda i: (i, 0)),
    ],
    out_shape=[
        jax.ShapeDtypeStruct((N, CW), _F32),
        jax.ShapeDtypeStruct((N, CW), _F32),
    ],
)


def _l45_body(aggp, glo, ghi, dinv, b3, wf1, bf1, wf2, bf2, out_o):
    gpre = jnp.concatenate([glo[...], ghi[...]], axis=1)
    u = (aggp[0] + aggp[1] + gpre) * dinv[...]
    h3 = jnp.maximum(u + b3[...], 0.0)
    h4 = jnp.maximum(
        jnp.dot(h3, wf1[...], preferred_element_type=_F32) + bf1[...], 0.0)
    z = jnp.dot(h4, wf2[...], preferred_element_type=_F32) + bf2[...]
    out_o[...] = 1.0 / (1.0 + jnp.exp(-z))


_l45_call = pl.pallas_call(
    _l45_body,
    grid=(N // BI,),
    in_specs=[
        pl.BlockSpec((NC, BI, 256), lambda i: (0, i, 0)),
        pl.BlockSpec((BI, CW), lambda i: (i, 0)),
        pl.BlockSpec((BI, CW), lambda i: (i, 0)),
        pl.BlockSpec((BI, 1), lambda i: (i, 0)),
        pl.BlockSpec((1, 256), lambda i: (0, 0)),
        pl.BlockSpec((256, 1024), lambda i: (0, 0)),
        pl.BlockSpec((1, 1024), lambda i: (0, 0)),
        pl.BlockSpec((1024, 1), lambda i: (0, 0)),
        pl.BlockSpec((1, 1), lambda i: (0, 0)),
    ],
    out_specs=pl.BlockSpec((BI, 1), lambda i: (i, 0)),
    out_shape=jax.ShapeDtypeStruct((N, 1), _F32),
)


def kernel(x, edge_index, W1, b1, W2, b2, W3, b3, Wfc1, bfc1, Wfc2, bfc2):
    ei = edge_index.astype(jnp.int32)
    src2d = ei[0].reshape(NR, B)
    dst2d = ei[1].reshape(NR, B)

    degp = _deg(dst2d)
    dinv, gpre1 = _p_call(degp, x)
    agg1 = _agg1(src2d, dst2d, gpre1)
    g2lo, g2hi = _l1_call(agg1, gpre1, dinv, W1, b1.reshape(1, -1))
    agg2 = _agg2(src2d, dst2d, g2lo, g2hi)
    g3lo, g3hi = _l23_call(agg2, g2lo, g2hi, dinv, W2, b2.reshape(1, -1), W3)
    agg3 = _agg2(src2d, dst2d, g3lo, g3hi)
    out = _l45_call(agg3, g3lo, g3hi, dinv, b3.reshape(1, -1),
                    Wfc1, bfc1.reshape(1, -1), Wfc2, bfc2.reshape(1, -1))
    return out


# speedup vs baseline: 1.0129x; 1.0129x over previous
import functools

import jax
import jax.numpy as jnp
from jax import lax
from jax.experimental import pallas as pl
from jax.experimental.pallas import tpu as pltpu
from jax.experimental.pallas import tpu_sc as plsc

N = 10000
NP = 10240
E = 320000
NC, NS = 2, 16
B = 125
RPT = E // (NC * NS * B)
NPT = NP // NS
CW = 128
ZR = 128
ZRA = 32
SUB = 16
BI = 1000
_F32 = jnp.float32

_mesh = plsc.VectorSubcoreMesh(core_axis_name="c", subcore_axis_name="s")


@functools.partial(
    pl.kernel,
    out_type=jax.ShapeDtypeStruct((NC, NP, 16), _F32),
    mesh=_mesh,
    scratch_types=[
        pltpu.VMEM((RPT, B), jnp.int32),
        pltpu.VMEM((B, 16), _F32),
        pltpu.VMEM((ZR, 16), _F32),
        pltpu.VMEM_SHARED((NP, 16), _F32),
    ],
)
def _deg(dst_hbm, out_hbm, dstv, onesv, zv, acc):
    cid = lax.axis_index("c")
    sid = lax.axis_index("s")
    base = (cid * NS + sid) * RPT
    pltpu.sync_copy(dst_hbm.at[pl.ds(base, RPT)], dstv)

    @pl.loop(0, B)
    def _(r):
        onesv[r, :] = jnp.ones((16,), _F32)

    @pl.loop(0, ZR)
    def _(r):
        zv[r, :] = jnp.zeros((16,), _F32)

    @pl.loop(0, NPT // ZR)
    def _(z):
        pltpu.sync_copy(zv, acc.at[pl.ds(sid * NPT + z * ZR, ZR)])

    plsc.subcore_barrier()

    @pl.loop(0, RPT)
    def _(r):
        pltpu.sync_copy(onesv, acc.at[dstv.at[r]], add=True)

    plsc.subcore_barrier()
    pltpu.sync_copy(acc.at[pl.ds(sid * NPT, NPT)],
                    out_hbm.at[cid, pl.ds(sid * NPT, NPT)])


def _make_agg(n_passes):
    @functools.partial(
        pl.kernel,
        out_type=jax.ShapeDtypeStruct((NC, NP, n_passes * CW), _F32),
        mesh=_mesh,
        scratch_types=[
            pltpu.VMEM((SUB, B), jnp.int32),
            pltpu.VMEM((SUB, B), jnp.int32),
            pltpu.VMEM((B, CW), _F32),
            pltpu.VMEM((B, CW), _F32),
            pltpu.VMEM((ZRA, CW), _F32),
            pltpu.VMEM_SHARED((NP, CW), _F32),
            pltpu.SemaphoreType.DMA,
            pltpu.SemaphoreType.DMA,
        ],
    )
    def agg(src_hbm, dst_hbm, *rest):
        gs = rest[:n_passes]
        out_hbm = rest[n_passes]
        srcv, dstv, buf0, buf1, zv, acc, sem0, sem1 = rest[n_passes + 1:]
        cid = lax.axis_index("c")
        sid = lax.axis_index("s")
        base = (cid * NS + sid) * RPT

        @pl.loop(0, ZRA)
        def _(r):
            @pl.loop(0, CW // 16)
            def _(c):
                zv[r, pl.ds(c * 16, 16)] = jnp.zeros((16,), _F32)

        for p, g_hbm in enumerate(gs):
            @pl.loop(0, NPT // ZRA)
            def _(z):
                pltpu.sync_copy(zv, acc.at[pl.ds(sid * NPT + z * ZRA, ZRA)])

            plsc.subcore_barrier()

            @pl.loop(0, RPT // SUB)
            def _(s):
                sbase = base + s * SUB
                pltpu.sync_copy(src_hbm.at[pl.ds(sbase, SUB)], srcv)
                pltpu.sync_copy(dst_hbm.at[pl.ds(sbase, SUB)], dstv)
                pltpu.async_copy(g_hbm.at[srcv.at[0]], buf0, sem0)

                @pl.loop(0, SUB, step=2)
                def _(i):
                    pltpu.make_async_copy(
                        g_hbm.at[srcv.at[i]], buf0, sem0).wait()
                    pltpu.async_copy(g_hbm.at[srcv.at[i + 1]], buf1, sem1)
                    pltpu.sync_copy(buf0, acc.at[dstv.at[i]], add=True)
                    pltpu.make_async_copy(
                        g_hbm.at[srcv.at[i + 1]], buf1, sem1).wait()

                    @pl.when(i + 2 < SUB)
                    def _():
                        pltpu.async_copy(g_hbm.at[srcv.at[i + 2]], buf0, sem0)

                    pltpu.sync_copy(buf1, acc.at[dstv.at[i + 1]], add=True)

            plsc.subcore_barrier()
            pltpu.sync_copy(
                acc.at[pl.ds(sid * NPT, NPT)],
                out_hbm.at[cid, pl.ds(sid * NPT, NPT), pl.ds(p * CW, CW)])

    return agg


_agg1 = _make_agg(1)
_agg2 = _make_agg(2)


def _p_body(degp, x, dinv_o, gpre_o):
    deg = degp[0, :, 0] + degp[1, :, 0] + 1.0
    dinv = lax.rsqrt(deg)
    dinv_o[...] = dinv[:, None]
    gpre_o[...] = x[...] * dinv[:, None]


_p_call = pl.pallas_call(
    _p_body,
    grid=(N // BI,),
    in_specs=[
        pl.BlockSpec((NC, BI, 16), lambda i: (0, i, 0)),
        pl.BlockSpec((BI, 128), lambda i: (i, 0)),
    ],
    out_specs=[
        pl.BlockSpec((BI, 1), lambda i: (i, 0)),
        pl.BlockSpec((BI, 128), lambda i: (i, 0)),
    ],
    out_shape=[
        jax.ShapeDtypeStruct((N, 1), _F32),
        jax.ShapeDtypeStruct((N, 128), _F32),
    ],
)


def _l1_body(aggp, gpre, dinv, w, b, lo_o, hi_o):
    u = (aggp[0] + aggp[1] + gpre[...]) * dinv[...]
    h = jnp.maximum(
        jnp.dot(u, w[...], preferred_element_type=_F32) + b[...], 0.0)
    g2 = h * dinv[...]
    lo_o[...] = g2[:, :CW]
    hi_o[...] = g2[:, CW:]


_l1_call = pl.pallas_call(
    _l1_body,
    grid=(N // BI,),
    in_specs=[
        pl.BlockSpec((NC, BI, 128), lambda i: (0, i, 0)),
        pl.BlockSpec((BI, 128), lambda i: (i, 0)),
        pl.BlockSpec((BI, 1), lambda i: (i, 0)),
        pl.BlockSpec((128, 256), lambda i: (0, 0)),
        pl.BlockSpec((1, 256), lambda i: (0, 0)),
    ],
    out_specs=[
        pl.BlockSpec((BI, CW), lambda i: (i, 0)),
        pl.BlockSpec((BI, CW), lambda i: (i, 0)),
    ],
    out_shape=[
        jax.ShapeDtypeStruct((N, CW), _F32),
        jax.ShapeDtypeStruct((N, CW), _F32),
    ],
)


def _l23_body(aggp, glo, ghi, dinv, w2, b2, w3, lo_o, hi_o):
    gpre = jnp.concatenate([glo[...], ghi[...]], axis=1)
    u = (aggp[0] + aggp[1] + gpre) * dinv[...]
    h2 = jnp.maximum(
        jnp.dot(u, w2[...], preferred_element_type=_F32) + b2[...], 0.0)
    g3 = jnp.dot(h2, w3[...], preferred_element_type=_F32) * dinv[...]
    lo_o[...] = g3[:, :CW]
    hi_o[...] = g3[:, CW:]


_l23_call = pl.pallas_call(
    _l23_body,
    grid=(N // BI,),
    in_specs=[
        pl.BlockSpec((NC, BI, 256), lambda i: (0, i, 0)),
        pl.BlockSpec((BI, CW), lambda i: (i, 0)),
        pl.BlockSpec((BI, CW), lambda i: (i, 0)),
        pl.BlockSpec((BI, 1), lambda i: (i, 0)),
        pl.BlockSpec((256, 512), lambda i: (0, 0)),
        pl.BlockSpec((1, 512), lambda i: (0, 0)),
        pl.BlockSpec((512, 256), lambda i: (0, 0)),
    ],
    out_specs=[
        pl.BlockSpec((BI, CW), lambda i: (i, 0)),
        pl.BlockSpec((BI, CW), lambda i: (i, 0)),
    ],
    out_shape=[
        jax.ShapeDtypeStruct((N, CW), _F32),
        jax.ShapeDtypeStruct((N, CW), _F32),
    ],
)


def _l45_body(aggp, glo, ghi, dinv, b3, wf1, bf1, wf2, bf2, out_o):
    gpre = jnp.concatenate([glo[...], ghi[...]], axis=1)
    u = (aggp[0] + aggp[1] + gpre) * dinv[...]
    h3 = jnp.maximum(u + b3[...], 0.0)
    h4 = jnp.maximum(
        jnp.dot(h3, wf1[...], preferred_element_type=_F32) + bf1[...], 0.0)
    z = jnp.dot(h4, wf2[...], preferred_element_type=_F32) + bf2[...]
    out_o[...] = 1.0 / (1.0 + jnp.exp(-z))


_l45_call = pl.pallas_call(
    _l45_body,
    grid=(N // BI,),
    in_specs=[
        pl.BlockSpec((NC, BI, 256), lambda i: (0, i, 0)),
        pl.BlockSpec((BI, CW), lambda i: (i, 0)),
        pl.BlockSpec((BI, CW), lambda i: (i, 0)),
        pl.BlockSpec((BI, 1), lambda i: (i, 0)),
        pl.BlockSpec((1, 256), lambda i: (0, 0)),
        pl.BlockSpec((256, 1024), lambda i: (0, 0)),
        pl.BlockSpec((1, 1024), lambda i: (0, 0)),
        pl.BlockSpec((1024, 1), lambda i: (0, 0)),
        pl.BlockSpec((1, 1), lambda i: (0, 0)),
    ],
    out_specs=pl.BlockSpec((BI, 1), lambda i: (i, 0)),
    out_shape=jax.ShapeDtypeStruct((N, 1), _F32),
)


def kernel(x, edge_index, W1, b1, W2, b2, W3, b3, Wfc1, bfc1, Wfc2, bfc2):
    ei = edge_index.astype(jnp.int32)
    src2d = ei[0].reshape(E // B, B)
    dst2d = ei[1].reshape(E // B, B)

    degp = _deg(dst2d)
    dinv, gpre1 = _p_call(degp, x)
    agg1 = _agg1(src2d, dst2d, gpre1)
    g2lo, g2hi = _l1_call(agg1, gpre1, dinv, W1, b1.reshape(1, -1))
    agg2 = _agg2(src2d, dst2d, g2lo, g2hi)
    g3lo, g3hi = _l23_call(agg2, g2lo, g2hi, dinv, W2, b2.reshape(1, -1), W3)
    agg3 = _agg2(src2d, dst2d, g3lo, g3hi)
    out = _l45_call(agg3, g3lo, g3hi, dinv, b3.reshape(1, -1),
                    Wfc1, bfc1.reshape(1, -1), Wfc2, bfc2.reshape(1, -1))
    return out


# speedup vs baseline: 1.0177x; 1.0047x over previous
import functools

import jax
import jax.numpy as jnp
from jax import lax
from jax.experimental import pallas as pl
from jax.experimental.pallas import tpu as pltpu
from jax.experimental.pallas import tpu_sc as plsc

N = 10000
NP = 10240
E = 320000
NC, NS = 2, 16
B = 125
RPT = E // (NC * NS * B)
NPT = NP // NS
CW = 128
ZR = 128
ZRA = 64
SUB = 16
BI = 1000
_F32 = jnp.float32

_mesh = plsc.VectorSubcoreMesh(core_axis_name="c", subcore_axis_name="s")


@functools.partial(
    pl.kernel,
    out_type=jax.ShapeDtypeStruct((NC, NP, 16), _F32),
    mesh=_mesh,
    scratch_types=[
        pltpu.VMEM((RPT, B), jnp.int32),
        pltpu.VMEM((B, 16), _F32),
        pltpu.VMEM((ZR, 16), _F32),
        pltpu.VMEM_SHARED((NP, 16), _F32),
        [pltpu.SemaphoreType.DMA] * 4,
    ],
)
def _deg(dst_hbm, out_hbm, dstv, onesv, zv, acc, sems):
    cid = lax.axis_index("c")
    sid = lax.axis_index("s")
    base = (cid * NS + sid) * RPT
    pltpu.sync_copy(dst_hbm.at[pl.ds(base, RPT)], dstv)

    @pl.loop(0, B)
    def _(r):
        onesv[r, :] = jnp.ones((16,), _F32)

    @pl.loop(0, ZR)
    def _(r):
        zv[r, :] = jnp.zeros((16,), _F32)

    @pl.loop(0, NPT // ZR)
    def _(z):
        pltpu.sync_copy(zv, acc.at[pl.ds(sid * NPT + z * ZR, ZR)])

    plsc.subcore_barrier()

    @pl.loop(0, RPT, step=4)
    def _(r):
        ds = [pltpu.async_copy(onesv, acc.at[dstv.at[r + j]], sems[j],
                               add=True) for j in range(4)]
        for d in ds:
            d.wait()

    plsc.subcore_barrier()
    pltpu.sync_copy(acc.at[pl.ds(sid * NPT, NPT)],
                    out_hbm.at[cid, pl.ds(sid * NPT, NPT)])


def _make_agg(n_passes):
    @functools.partial(
        pl.kernel,
        out_type=jax.ShapeDtypeStruct((NC, NP, n_passes * CW), _F32),
        mesh=_mesh,
        scratch_types=[
            pltpu.VMEM((SUB, B), jnp.int32),
            pltpu.VMEM((SUB, B), jnp.int32),
            pltpu.VMEM((B, CW), _F32),
            pltpu.VMEM((B, CW), _F32),
            pltpu.VMEM((ZRA, CW), _F32),
            pltpu.VMEM_SHARED((NP, CW), _F32),
            pltpu.SemaphoreType.DMA,
            pltpu.SemaphoreType.DMA,
        ],
    )
    def agg(src_hbm, dst_hbm, *rest):
        gs = rest[:n_passes]
        out_hbm = rest[n_passes]
        srcv, dstv, buf0, buf1, zv, acc, sem0, sem1 = rest[n_passes + 1:]
        cid = lax.axis_index("c")
        sid = lax.axis_index("s")
        base = (cid * NS + sid) * RPT

        @pl.loop(0, ZRA)
        def _(r):
            @pl.loop(0, CW // 16)
            def _(c):
                zv[r, pl.ds(c * 16, 16)] = jnp.zeros((16,), _F32)

        for p, g_hbm in enumerate(gs):
            @pl.loop(0, NPT // ZRA)
            def _(z):
                pltpu.sync_copy(zv, acc.at[pl.ds(sid * NPT + z * ZRA, ZRA)])

            plsc.subcore_barrier()

            @pl.loop(0, RPT // SUB)
            def _(s):
                sbase = base + s * SUB
                pltpu.sync_copy(src_hbm.at[pl.ds(sbase, SUB)], srcv)
                pltpu.sync_copy(dst_hbm.at[pl.ds(sbase, SUB)], dstv)
                pltpu.async_copy(g_hbm.at[srcv.at[0]], buf0, sem0)

                @pl.loop(0, SUB, step=2)
                def _(i):
                    pltpu.make_async_copy(
                        g_hbm.at[srcv.at[i]], buf0, sem0).wait()
                    pltpu.async_copy(g_hbm.at[srcv.at[i + 1]], buf1, sem1)
                    pltpu.sync_copy(buf0, acc.at[dstv.at[i]], add=True)
                    pltpu.make_async_copy(
                        g_hbm.at[srcv.at[i + 1]], buf1, sem1).wait()

                    @pl.when(i + 2 < SUB)
                    def _():
                        pltpu.async_copy(g_hbm.at[srcv.at[i + 2]], buf0, sem0)

                    pltpu.sync_copy(buf1, acc.at[dstv.at[i + 1]], add=True)

            plsc.subcore_barrier()
            pltpu.sync_copy(
                acc.at[pl.ds(sid * NPT, NPT)],
                out_hbm.at[cid, pl.ds(sid * NPT, NPT), pl.ds(p * CW, CW)])

    return agg


_agg1 = _make_agg(1)
_agg2 = _make_agg(2)


def _p_body(degp, x, dinv_o, gpre_o):
    deg = degp[0, :, 0] + degp[1, :, 0] + 1.0
    dinv = lax.rsqrt(deg)
    dinv_o[...] = dinv[:, None]
    gpre_o[...] = x[...] * dinv[:, None]


_p_call = pl.pallas_call(
    _p_body,
    grid=(N // BI,),
    in_specs=[
        pl.BlockSpec((NC, BI, 16), lambda i: (0, i, 0)),
        pl.BlockSpec((BI, 128), lambda i: (i, 0)),
    ],
    out_specs=[
        pl.BlockSpec((BI, 1), lambda i: (i, 0)),
        pl.BlockSpec((BI, 128), lambda i: (i, 0)),
    ],
    out_shape=[
        jax.ShapeDtypeStruct((N, 1), _F32),
        jax.ShapeDtypeStruct((N, 128), _F32),
    ],
)


def _l1_body(aggp, gpre, dinv, w, b, lo_o, hi_o):
    u = (aggp[0] + aggp[1] + gpre[...]) * dinv[...]
    h = jnp.maximum(
        jnp.dot(u, w[...], preferred_element_type=_F32) + b[...], 0.0)
    g2 = h * dinv[...]
    lo_o[...] = g2[:, :CW]
    hi_o[...] = g2[:, CW:]


_l1_call = pl.pallas_call(
    _l1_body,
    grid=(N // BI,),
    in_specs=[
        pl.BlockSpec((NC, BI, 128), lambda i: (0, i, 0)),
        pl.BlockSpec((BI, 128), lambda i: (i, 0)),
        pl.BlockSpec((BI, 1), lambda i: (i, 0)),
        pl.BlockSpec((128, 256), lambda i: (0, 0)),
        pl.BlockSpec((1, 256), lambda i: (0, 0)),
    ],
    out_specs=[
        pl.BlockSpec((BI, CW), lambda i: (i, 0)),
        pl.BlockSpec((BI, CW), lambda i: (i, 0)),
    ],
    out_shape=[
        jax.ShapeDtypeStruct((N, CW), _F32),
        jax.ShapeDtypeStruct((N, CW), _F32),
    ],
)


def _l23_body(aggp, glo, ghi, dinv, w2, b2, w3, lo_o, hi_o):
    gpre = jnp.concatenate([glo[...], ghi[...]], axis=1)
    u = (aggp[0] + aggp[1] + gpre) * dinv[...]
    h2 = jnp.maximum(
        jnp.dot(u, w2[...], preferred_element_type=_F32) + b2[...], 0.0)
    g3 = jnp.dot(h2, w3[...], preferred_element_type=_F32) * dinv[...]
    lo_o[...] = g3[:, :CW]
    hi_o[...] = g3[:, CW:]


_l23_call = pl.pallas_call(
    _l23_body,
    grid=(N // BI,),
    in_specs=[
        pl.BlockSpec((NC, BI, 256), lambda i: (0, i, 0)),
        pl.BlockSpec((BI, CW), lambda i: (i, 0)),
        pl.BlockSpec((BI, CW), lambda i: (i, 0)),
        pl.BlockSpec((BI, 1), lambda i: (i, 0)),
        pl.BlockSpec((256, 512), lambda i: (0, 0)),
        pl.BlockSpec((1, 512), lambda i: (0, 0)),
        pl.BlockSpec((512, 256), lambda i: (0, 0)),
    ],
    out_specs=[
        pl.BlockSpec((BI, CW), lambda i: (i, 0)),
        pl.BlockSpec((BI, CW), lambda i: (i, 0)),
    ],
    out_shape=[
        jax.ShapeDtypeStruct((N, CW), _F32),
        jax.ShapeDtypeStruct((N, CW), _F32),
    ],
)


def _l45_body(aggp, glo, ghi, dinv, b3, wf1, bf1, wf2, bf2, out_o):
    gpre = jnp.concatenate([glo[...], ghi[...]], axis=1)
    u = (aggp[0] + aggp[1] + gpre) * dinv[...]
    h3 = jnp.maximum(u + b3[...], 0.0)
    h4 = jnp.maximum(
        jnp.dot(h3, wf1[...], preferred_element_type=_F32) + bf1[...], 0.0)
    z = jnp.dot(h4, wf2[...], preferred_element_type=_F32) + bf2[...]
    out_o[...] = 1.0 / (1.0 + jnp.exp(-z))


_l45_call = pl.pallas_call(
    _l45_body,
    grid=(N // BI,),
    in_specs=[
        pl.BlockSpec((NC, BI, 256), lambda i: (0, i, 0)),
        pl.BlockSpec((BI, CW), lambda i: (i, 0)),
        pl.BlockSpec((BI, CW), lambda i: (i, 0)),
        pl.BlockSpec((BI, 1), lambda i: (i, 0)),
        pl.BlockSpec((1, 256), lambda i: (0, 0)),
        pl.BlockSpec((256, 1024), lambda i: (0, 0)),
        pl.BlockSpec((1, 1024), lambda i: (0, 0)),
        pl.BlockSpec((1024, 1), lambda i: (0, 0)),
        pl.BlockSpec((1, 1), lambda i: (0, 0)),
    ],
    out_specs=pl.BlockSpec((BI, 1), lambda i: (i, 0)),
    out_shape=jax.ShapeDtypeStruct((N, 1), _F32),
)


def kernel(x, edge_index, W1, b1, W2, b2, W3, b3, Wfc1, bfc1, Wfc2, bfc2):
    ei = edge_index.astype(jnp.int32)
    src2d = ei[0].reshape(E // B, B)
    dst2d = ei[1].reshape(E // B, B)

    degp = _deg(dst2d)
    dinv, gpre1 = _p_call(degp, x)
    agg1 = _agg1(src2d, dst2d, gpre1)
    g2lo, g2hi = _l1_call(agg1, gpre1, dinv, W1, b1.reshape(1, -1))
    agg2 = _agg2(src2d, dst2d, g2lo, g2hi)
    g3lo, g3hi = _l23_call(agg2, g2lo, g2hi, dinv, W2, b2.reshape(1, -1), W3)
    agg3 = _agg2(src2d, dst2d, g3lo, g3hi)
    out = _l45_call(agg3, g3lo, g3hi, dinv, b3.reshape(1, -1),
                    Wfc1, bfc1.reshape(1, -1), Wfc2, bfc2.reshape(1, -1))
    return out
